# Initial kernel scaffold; baseline (speedup 1.0000x reference)
import jax
import jax.numpy as jnp
from jax.experimental import pallas as pl


def kernel(inputs):
    raise NotImplementedError("write your pallas kernel here")


# speedup vs baseline: 3.6659x; 3.6659x over previous
import jax
import jax.numpy as jnp
from jax.experimental import pallas as pl

_M, _N = 16384, 256
_BM = 1024


def _body(in_ref, out_ref):
    a = in_ref[:, : _N // 2]
    b = in_ref[:, _N // 2 :]
    lane = jax.lax.broadcasted_iota(jnp.int32, (_BM, _N // 2), 1)
    idx = (lane * 2) % (_N // 2)
    ga = jnp.take_along_axis(a, idx, axis=1)
    gb = jnp.take_along_axis(b, idx, axis=1)
    out_ref[...] = jnp.where(lane < _N // 4, ga, gb)


def kernel(inputs):
    return pl.pallas_call(
        _body,
        grid=(_M // _BM,),
        in_specs=[pl.BlockSpec((_BM, _N), lambda i: (i, 0))],
        out_specs=pl.BlockSpec((_BM, _N // 2), lambda i: (i, 0)),
        out_shape=jax.ShapeDtypeStruct((_M, _N // 2), jnp.float32),
    )(inputs)
